# baseline (speedup 1.0000x reference)
import functools

import jax
import jax.numpy as jnp
from jax import lax
from jax.experimental import pallas as pl
from jax.experimental.pallas import tpu as pltpu
from jax.experimental.pallas import tpu_sc as plsc

NUM_ENTITIES = 1000000
NUM_RELATIONS = 1000
EMBED_DIM = 64
BATCH = 16384

NC = 2
NS = 16
NW = NC * NS
N_ENT = 2 * BATCH // NW
N_REL = BATCH // NW
PASS = 256
RING = 3
SENTINEL = 0x7FFFFFF0


def _sca(ref, i):
    v = plsc.load_gather(ref, [jnp.full((16,), i, dtype=jnp.int32)])
    return v[0]


def _gather3_kernel(ss_hbm, sp_hbm, r_hbm, entT_hbm, relT_hbm,
                    h_out, r_out, t_out,
                    ss_v, sp_v, r_v,
                    blockbuf, staging, relbuf,
                    sem_blk, sem_out, sem_rel):
    wid = lax.axis_index("s") * NC + lax.axis_index("c")
    lanes = lax.iota(jnp.int32, 16)

    rel_cp = pltpu.async_copy(relT_hbm.at[:, pl.ds(0, 1024)], relbuf, sem_rel)

    pltpu.sync_copy(ss_hbm.at[pl.ds(wid * N_ENT, N_ENT)], ss_v.at[pl.ds(0, N_ENT)])
    pltpu.sync_copy(sp_hbm.at[pl.ds(wid * N_ENT, N_ENT)], sp_v)
    pltpu.sync_copy(r_hbm.at[pl.ds(wid * N_REL, N_REL)], r_v)
    ss_v[pl.ds(N_ENT, 16)] = jnp.full((16,), SENTINEL, dtype=jnp.int32)

    def run_pass(src_hbm, sidx_v, row0, n_rows):
        b0 = _sca(sidx_v, row0) >> 7
        b1 = _sca(sidx_v, row0 + n_rows - 1) >> 7
        nblocks = b1 - b0 + 1

        def fetch(b, slot):
            bb = jnp.minimum(b, b1)
            pltpu.async_copy(
                src_hbm.at[:, pl.ds(bb * 128, 128)],
                blockbuf.at[slot], sem_blk)

        for j in range(RING):
            fetch(b0 + j, j % RING)

        def block_body(bi, i):
            pltpu.make_async_copy(src_hbm.at[:, pl.ds(0, 128)],
                                  blockbuf.at[0], sem_blk).wait()
            slot = lax.rem(bi, RING)
            cur = b0 + bi

            def row_cond(i2):
                return jnp.logical_and(
                    i2 < row0 + n_rows, (_sca(sidx_v, i2) >> 7) == cur)

            def row_body(i2):
                s16 = plsc.load_gather(
                    sidx_v, [jnp.full((16,), i2, dtype=jnp.int32)])
                lane16 = jnp.bitwise_and(s16, 127)
                slot16 = jnp.full((16,), slot, dtype=jnp.int32)
                for g in range(EMBED_DIM // 16):
                    vals = plsc.load_gather(
                        blockbuf, [slot16, lanes + g * 16, lane16])
                    staging[i2 - row0, pl.ds(g * 16, 16)] = vals
                return i2 + 1

            i = lax.while_loop(row_cond, row_body, i)
            fetch(b0 + bi + RING, slot)
            return i

        lax.fori_loop(0, nblocks, block_body, row0, unroll=False)
        pltpu.make_async_copy(src_hbm.at[:, pl.ds(0, 128 * RING)],
                              blockbuf, sem_blk).wait()

    def scatter_ent(row0):
        def body(i, _):
            p = _sca(sp_v, row0 + i)
            pos = jnp.bitwise_and(p, BATCH - 1)
            tb = p >> 14

            @pl.when(tb == 0)
            def _():
                pltpu.async_copy(staging.at[pl.ds(i, 1)],
                                 h_out.at[pl.ds(pos, 1)], sem_out)

            @pl.when(tb == 1)
            def _():
                pltpu.async_copy(staging.at[pl.ds(i, 1)],
                                 t_out.at[pl.ds(pos, 1)], sem_out)
            return 0

        lax.fori_loop(0, PASS, body, 0, unroll=False)
        pltpu.make_async_copy(h_out.at[pl.ds(0, PASS)],
                              staging, sem_out).wait()

    for row0 in range(0, N_ENT, PASS):
        run_pass(entT_hbm, ss_v, row0, PASS)
        scatter_ent(row0)

    rel_cp.wait()
    for row0 in range(0, N_REL, PASS):
        def rel_body(i, _, row0=row0):
            s16 = plsc.load_gather(
                r_v, [jnp.full((16,), row0 + i, dtype=jnp.int32)])
            for g in range(EMBED_DIM // 16):
                vals = plsc.load_gather(relbuf, [lanes + g * 16, s16])
                staging[i, pl.ds(g * 16, 16)] = vals
            return 0

        lax.fori_loop(0, PASS, rel_body, 0, unroll=False)
        pltpu.async_copy(staging,
                         r_out.at[pl.ds(wid * N_REL + row0, PASS)], sem_out)
        pltpu.make_async_copy(h_out.at[pl.ds(0, PASS)],
                              staging, sem_out).wait()


@jax.jit
def _gather3(h, r, t, ent_embeddings, rel_embeddings):
    entT = ent_embeddings.T
    relT = rel_embeddings.T
    pos = jnp.arange(2 * BATCH, dtype=jnp.int32)
    ss, sp = lax.sort((jnp.concatenate([h, t]), pos), num_keys=1)

    mesh = plsc.VectorSubcoreMesh(core_axis_name="c", subcore_axis_name="s")
    out = jax.ShapeDtypeStruct((BATCH, EMBED_DIM), jnp.float32)
    run = pl.kernel(
        _gather3_kernel,
        mesh=mesh,
        compiler_params=pltpu.CompilerParams(
            disable_bounds_checks=True, disable_semaphore_checks=True,
            needs_layout_passes=False),
        out_type=(out, out, out),
        scratch_types=[
            pltpu.VMEM((N_ENT + 16,), jnp.int32),
            pltpu.VMEM((N_ENT,), jnp.int32),
            pltpu.VMEM((N_REL,), jnp.int32),
            pltpu.VMEM((RING, EMBED_DIM, 128), jnp.float32),
            pltpu.VMEM((PASS, EMBED_DIM), jnp.float32),
            pltpu.VMEM((EMBED_DIM, 1024), jnp.float32),
            pltpu.SemaphoreType.DMA,
            pltpu.SemaphoreType.DMA,
            pltpu.SemaphoreType.DMA,
        ],
    )
    return run(ss, sp, r, entT, relT)


def kernel(h, r, t, ent_embeddings, rel_embeddings):
    return _gather3(h.astype(jnp.int32), r.astype(jnp.int32),
                    t.astype(jnp.int32), ent_embeddings, rel_embeddings)


# speedup vs baseline: 1.0367x; 1.0367x over previous
import functools

import jax
import jax.numpy as jnp
from jax import lax
from jax.experimental import pallas as pl
from jax.experimental.pallas import tpu as pltpu
from jax.experimental.pallas import tpu_sc as plsc

NUM_ENTITIES = 1000000
NUM_RELATIONS = 1000
EMBED_DIM = 64
BATCH = 16384

NC = 2
NS = 16
NW = NC * NS
N_ENT = 2 * BATCH // NW
N_REL = BATCH // NW
PASS = 256
RING = 8
SENTINEL = 0x7FFFFFF0


def _sca(ref, i):
    v = plsc.load_gather(ref, [jnp.full((16,), i, dtype=jnp.int32)])
    return v[0]


def _gather3_kernel(ss_hbm, sp_hbm, rs_hbm, rp_hbm, entT_hbm, relT_hbm,
                    h_out, r_out, t_out,
                    ss_v, sp_v, rs_v, rp_v,
                    blockbuf, staging,
                    sem_blk, sem_out):
    wid = lax.axis_index("s") * NC + lax.axis_index("c")
    lanes = lax.iota(jnp.int32, 16)

    pltpu.sync_copy(ss_hbm.at[pl.ds(wid * N_ENT, N_ENT)], ss_v.at[pl.ds(0, N_ENT)])
    pltpu.sync_copy(sp_hbm.at[pl.ds(wid * N_ENT, N_ENT)], sp_v)
    pltpu.sync_copy(rs_hbm.at[pl.ds(wid * N_REL, N_REL)], rs_v.at[pl.ds(0, N_REL)])
    pltpu.sync_copy(rp_hbm.at[pl.ds(wid * N_REL, N_REL)], rp_v)
    ss_v[pl.ds(N_ENT, 16)] = jnp.full((16,), SENTINEL, dtype=jnp.int32)
    rs_v[pl.ds(N_REL, 16)] = jnp.full((16,), SENTINEL, dtype=jnp.int32)

    def run_pass(src_hbm, sidx_v, row0, n_rows):
        b0 = _sca(sidx_v, row0) >> 7
        b1 = _sca(sidx_v, row0 + n_rows - 1) >> 7
        nblocks = b1 - b0 + 1

        def fetch(b, slot):
            bb = jnp.minimum(b, b1)
            pltpu.async_copy(
                src_hbm.at[:, pl.ds(bb * 128, 128)],
                blockbuf.at[slot], sem_blk)

        for j in range(RING):
            fetch(b0 + j, j % RING)

        def block_body(bi, i):
            pltpu.make_async_copy(src_hbm.at[:, pl.ds(0, 128)],
                                  blockbuf.at[0], sem_blk).wait()
            slot = lax.rem(bi, RING)
            cur = b0 + bi

            def row_cond(i2):
                return jnp.logical_and(
                    i2 < row0 + n_rows, (_sca(sidx_v, i2) >> 7) == cur)

            def row_body(i2):
                s16 = plsc.load_gather(
                    sidx_v, [jnp.full((16,), i2, dtype=jnp.int32)])
                lane16 = jnp.bitwise_and(s16, 127)
                slot16 = jnp.full((16,), slot, dtype=jnp.int32)
                for g in range(EMBED_DIM // 16):
                    vals = plsc.load_gather(
                        blockbuf, [slot16, lanes + g * 16, lane16])
                    staging[i2 - row0, pl.ds(g * 16, 16)] = vals
                return i2 + 1

            i = lax.while_loop(row_cond, row_body, i)
            fetch(b0 + bi + RING, slot)
            return i

        lax.fori_loop(0, nblocks, block_body, row0, unroll=False)
        pltpu.make_async_copy(src_hbm.at[:, pl.ds(0, 128 * RING)],
                              blockbuf, sem_blk).wait()

    def scatter_ent(row0):
        def body(i, _):
            p = _sca(sp_v, row0 + i)
            pos = jnp.bitwise_and(p, BATCH - 1)
            tb = p >> 14

            @pl.when(tb == 0)
            def _():
                pltpu.async_copy(staging.at[pl.ds(i, 1)],
                                 h_out.at[pl.ds(pos, 1)], sem_out)

            @pl.when(tb == 1)
            def _():
                pltpu.async_copy(staging.at[pl.ds(i, 1)],
                                 t_out.at[pl.ds(pos, 1)], sem_out)
            return 0

        lax.fori_loop(0, PASS, body, 0, unroll=False)
        pltpu.make_async_copy(h_out.at[pl.ds(0, PASS)],
                              staging, sem_out).wait()

    for row0 in range(0, N_ENT, PASS):
        run_pass(entT_hbm, ss_v, row0, PASS)
        scatter_ent(row0)

    for row0 in range(0, N_REL, PASS):
        run_pass(relT_hbm, rs_v, row0, PASS)

        def scatter_rel(i, _, row0=row0):
            pos = _sca(rp_v, row0 + i)
            pltpu.async_copy(staging.at[pl.ds(i, 1)],
                             r_out.at[pl.ds(pos, 1)], sem_out)
            return 0

        lax.fori_loop(0, PASS, scatter_rel, 0, unroll=False)
        pltpu.make_async_copy(h_out.at[pl.ds(0, PASS)],
                              staging, sem_out).wait()


@jax.jit
def _gather3(h, r, t, ent_embeddings, rel_embeddings):
    entT = ent_embeddings.T
    relT = rel_embeddings.T
    pos = jnp.arange(2 * BATCH, dtype=jnp.int32)
    ss, sp = lax.sort((jnp.concatenate([h, t]), pos), num_keys=1)
    rs, rp = lax.sort((r, jnp.arange(BATCH, dtype=jnp.int32)), num_keys=1)

    mesh = plsc.VectorSubcoreMesh(core_axis_name="c", subcore_axis_name="s")
    out = jax.ShapeDtypeStruct((BATCH, EMBED_DIM), jnp.float32)
    run = pl.kernel(
        _gather3_kernel,
        mesh=mesh,
        compiler_params=pltpu.CompilerParams(
            disable_bounds_checks=True, disable_semaphore_checks=True,
            needs_layout_passes=False),
        out_type=(out, out, out),
        scratch_types=[
            pltpu.VMEM((N_ENT + 16,), jnp.int32),
            pltpu.VMEM((N_ENT,), jnp.int32),
            pltpu.VMEM((N_REL + 16,), jnp.int32),
            pltpu.VMEM((N_REL,), jnp.int32),
            pltpu.VMEM((RING, EMBED_DIM, 128), jnp.float32),
            pltpu.VMEM((PASS, EMBED_DIM), jnp.float32),
            pltpu.SemaphoreType.DMA,
            pltpu.SemaphoreType.DMA,
        ],
    )
    return run(ss, sp, rs, rp, entT, relT)


def kernel(h, r, t, ent_embeddings, rel_embeddings):
    return _gather3(h.astype(jnp.int32), r.astype(jnp.int32),
                    t.astype(jnp.int32), ent_embeddings, rel_embeddings)
